# baseline (speedup 1.0000x reference)
import functools

import jax
import jax.numpy as jnp
from jax import lax
from jax.experimental import pallas as pl
from jax.experimental.pallas import tpu as pltpu
from jax.experimental.pallas import tpu_sc as plsc

BATCH = 16384
EMB_DIM = 32
LANES = 16

_INFO = plsc.get_sparse_core_info()
_NC = _INFO.num_cores
_NS = _INFO.num_subcores
NW = _NC * _NS
BPW = BATCH // NW
GROUPS = BPW // LANES
CW = 128
CH = BPW // CW


def _sc_body(u_idx_hbm, p_idx_hbm, n_idx_hbm, uemb_hbm, iemb_hbm, bias_hbm,
             pos_out, neg_out,
             uidx_v, pidx_v, nidx_v, ue_v, pe_v, ne_v, pb_v, nb_v,
             ps_v, ns_v, sem):
    wid = lax.axis_index("s") * _NC + lax.axis_index("c")
    base = wid * BPW

    pltpu.sync_copy(u_idx_hbm.at[pl.ds(base, BPW)], uidx_v)
    pltpu.sync_copy(p_idx_hbm.at[pl.ds(base, BPW)], pidx_v)
    pltpu.sync_copy(n_idx_hbm.at[pl.ds(base, BPW)], nidx_v)

    for c in range(CH):
        sl = pl.ds(c * CW, CW)
        pltpu.async_copy(uemb_hbm.at[uidx_v.at[sl]], ue_v.at[sl], sem)
        pltpu.async_copy(iemb_hbm.at[pidx_v.at[sl]], pe_v.at[sl], sem)
        pltpu.async_copy(iemb_hbm.at[nidx_v.at[sl]], ne_v.at[sl], sem)
        pltpu.async_copy(bias_hbm.at[pidx_v.at[sl]], pb_v.at[sl], sem)
        pltpu.async_copy(bias_hbm.at[nidx_v.at[sl]], nb_v.at[sl], sem)

    pltpu.make_async_copy(uemb_hbm.at[pl.ds(0, BPW)], ue_v, sem).wait()
    pltpu.make_async_copy(uemb_hbm.at[pl.ds(0, BPW)], pe_v, sem).wait()
    pltpu.make_async_copy(uemb_hbm.at[pl.ds(0, BPW)], ne_v, sem).wait()
    pltpu.make_async_copy(bias_hbm.at[pl.ds(0, BPW)], pb_v, sem).wait()
    pltpu.make_async_copy(bias_hbm.at[pl.ds(0, BPW)], nb_v, sem).wait()

    iot = lax.iota(jnp.int32, LANES)

    def grp(g, carry):
        rb = g * LANES
        rows = rb + iot
        accp = pb_v[pl.ds(rb, LANES)]
        accn = nb_v[pl.ds(rb, LANES)]
        for d in range(EMB_DIM):
            dsplat = jnp.full((LANES,), d, jnp.int32)
            uv = plsc.load_gather(ue_v, [rows, dsplat])
            pv = plsc.load_gather(pe_v, [rows, dsplat])
            nv = plsc.load_gather(ne_v, [rows, dsplat])
            accp = accp + uv * pv
            accn = accn + uv * nv
        ps_v[pl.ds(rb, LANES)] = accp
        ns_v[pl.ds(rb, LANES)] = accn
        return carry

    lax.fori_loop(0, GROUPS, grp, 0)

    pltpu.sync_copy(ps_v, pos_out.at[pl.ds(base, BPW)])
    pltpu.sync_copy(ns_v, neg_out.at[pl.ds(base, BPW)])


@jax.jit
def _bprmf_sc(u_idx, p_idx, n_idx, uemb, iemb, bias1):
    mesh = plsc.VectorSubcoreMesh(core_axis_name="c", subcore_axis_name="s")
    f = functools.partial(
        pl.kernel,
        mesh=mesh,
        out_type=(
            jax.ShapeDtypeStruct((BATCH,), jnp.float32),
            jax.ShapeDtypeStruct((BATCH,), jnp.float32),
        ),
        scratch_types=[
            pltpu.VMEM((BPW,), jnp.int32),
            pltpu.VMEM((BPW,), jnp.int32),
            pltpu.VMEM((BPW,), jnp.int32),
            pltpu.VMEM((BPW, EMB_DIM), jnp.float32),
            pltpu.VMEM((BPW, EMB_DIM), jnp.float32),
            pltpu.VMEM((BPW, EMB_DIM), jnp.float32),
            pltpu.VMEM((BPW,), jnp.float32),
            pltpu.VMEM((BPW,), jnp.float32),
            pltpu.VMEM((BPW,), jnp.float32),
            pltpu.VMEM((BPW,), jnp.float32),
            pltpu.SemaphoreType.DMA,
        ],
        compiler_params=pltpu.CompilerParams(
            use_tc_tiling_on_sc=False,
            needs_layout_passes=False,
        ),
    )(_sc_body)
    return f(u_idx, p_idx, n_idx, uemb, iemb, bias1)


def kernel(users, pos_items, neg_items, user_embedding, item_embedding, item_bias):
    return _bprmf_sc(
        users.astype(jnp.int32),
        pos_items.astype(jnp.int32),
        neg_items.astype(jnp.int32),
        user_embedding,
        item_embedding,
        item_bias.reshape(-1),
    )
